# baseline (speedup 1.0000x reference)
import functools

import jax
import jax.numpy as jnp
from jax import lax
from jax.experimental import pallas as pl
from jax.experimental.pallas import tpu as pltpu
from jax.experimental.pallas import tpu_sc as plsc

_NC, _NS = 2, 16
_NW = _NC * _NS
_CHUNK = 128
_RING = 6
_RING_S = 8
_NEG = -1e30


def _sc_mesh():
    return plsc.VectorSubcoreMesh(
        core_axis_name="c", subcore_axis_name="s",
        num_cores=_NC, num_subcores=_NS)


def _relu_mm2(x, W, b, tile, n_pad):
    M, K = x.shape
    O = W.shape[1]

    def body(x_ref, w_ref, b_ref, o64_ref, o128_ref):
        acc = jnp.dot(x_ref[...], w_ref[...], preferred_element_type=jnp.float32)
        v = jnp.maximum(acc + b_ref[...], 0.0)
        o64_ref[...] = v
        o128_ref[...] = jnp.concatenate(
            [v, jnp.zeros((tile, O), jnp.float32)], axis=1)

    return pl.pallas_call(
        body,
        grid=(M // tile,),
        in_specs=[pl.BlockSpec((tile, K), lambda i: (i, 0)),
                  pl.BlockSpec((K, O), lambda i: (0, 0)),
                  pl.BlockSpec((1, O), lambda i: (0, 0))],
        out_specs=[pl.BlockSpec((tile, O), lambda i: (i, 0)),
                   pl.BlockSpec((tile, 2 * O), lambda i: (i, 0))],
        out_shape=[jax.ShapeDtypeStruct((n_pad, O), jnp.float32),
                   jax.ShapeDtypeStruct((n_pad, 2 * O), jnp.float32)],
    )(x, W, b.reshape(1, O))


def _relu_mm_t(x, W, b, tile):
    M, K = x.shape
    O = W.shape[1]

    def body(x_ref, w_ref, b_ref, o_ref):
        acc = jnp.dot(x_ref[...], w_ref[...], preferred_element_type=jnp.float32)
        o_ref[...] = jnp.maximum(acc + b_ref[...], 0.0).T

    return pl.pallas_call(
        body,
        grid=(M // tile,),
        in_specs=[pl.BlockSpec((tile, K), lambda i: (i, 0)),
                  pl.BlockSpec((K, O), lambda i: (0, 0)),
                  pl.BlockSpec((1, O), lambda i: (0, 0))],
        out_specs=pl.BlockSpec((O, tile), lambda i: (0, i)),
        out_shape=jax.ShapeDtypeStruct((O, M), jnp.float32),
    )(x, W, b.reshape(1, O))


def _sc_gather(table, idx):
    n_rows, Hd = table.shape
    (E,) = idx.shape
    n_chunks = E // _CHUNK
    maxc = -(-n_chunks // _NW)

    @functools.partial(
        pl.kernel,
        out_type=jax.ShapeDtypeStruct((E, Hd), jnp.float32),
        mesh=_sc_mesh(),
        scratch_types=[
            [pltpu.VMEM((_CHUNK,), jnp.int32)] * _RING,
            pltpu.VMEM((_RING, _CHUNK, Hd), jnp.float32),
            [pltpu.SemaphoreType.DMA] * _RING,
        ] + [pltpu.SemaphoreType.DMA] * _RING,
    )
    def k(table_hbm, idx_hbm, out_hbm, idxb, rows, semi, *semg):
        wid = lax.axis_index("s") * _NC + lax.axis_index("c")
        base = (wid * n_chunks) // _NW
        cnt = ((wid + 1) * n_chunks) // _NW - base

        def fire_idx(j):
            pltpu.async_copy(
                idx_hbm.at[pl.ds((base + j) * _CHUNK, _CHUNK)],
                idxb[j % _RING], semi[j % _RING])

        def wait_idx(j):
            pltpu.make_async_copy(
                idx_hbm.at[pl.ds((base + j) * _CHUNK, _CHUNK)],
                idxb[j % _RING], semi[j % _RING]).wait()

        def fire_g(j):
            pltpu.async_copy(table_hbm.at[idxb[j % _RING]],
                             rows.at[j % _RING], semg[j % _RING])

        def wait_g(j):
            pltpu.make_async_copy(table_hbm.at[idxb[j % _RING]],
                                  rows.at[j % _RING], semg[j % _RING]).wait()

        for j in range(_RING):
            pl.when(j < cnt)(lambda j=j: fire_idx(j))
        for j in range(_RING - 1):
            @pl.when(j < cnt)
            def _(j=j):
                wait_idx(j)
                fire_g(j)
        for j in range(maxc):
            @pl.when(j < cnt)
            def _(j=j):
                wait_g(j)
                pltpu.sync_copy(rows.at[j % _RING],
                                out_hbm.at[pl.ds((base + j) * _CHUNK, _CHUNK), :])
                pl.when(j + _RING < cnt)(lambda j=j: fire_idx(j + _RING))

                @pl.when(j + _RING - 1 < cnt)
                def _():
                    wait_idx(j + _RING - 1)
                    fire_g(j + _RING - 1)

    return k(table, idx)


def _sc_scatter(msg, dst, zeros_rows):
    E, Hd = msg.shape
    n_acc = zeros_rows.shape[0]
    n_chunks = E // _CHUNK
    maxc = -(-n_chunks // _NW)
    band = n_acc // _NS

    @functools.partial(
        pl.kernel,
        out_type=jax.ShapeDtypeStruct((2 * n_acc, Hd), jnp.float32),
        mesh=_sc_mesh(),
        compiler_params=pltpu.CompilerParams(use_tc_tiling_on_sc=False),
        scratch_types=[
            [pltpu.VMEM((_CHUNK,), jnp.int32)] * _RING_S,
            pltpu.VMEM((_RING_S, _CHUNK, Hd), jnp.float32),
            pltpu.VMEM_SHARED((n_acc, Hd), jnp.float32),
            [pltpu.SemaphoreType.DMA] * _RING_S,
            [pltpu.SemaphoreType.DMA] * _RING_S,
        ] + [pltpu.SemaphoreType.DMA] * _RING_S,
    )
    def k(msg_hbm, dst_hbm, zero_hbm, out_hbm, idxb, bufs, acc,
          semi, sl, *ss):
        c = lax.axis_index("c")
        s = lax.axis_index("s")
        wid = s * _NC + c
        base = (wid * n_chunks) // _NW
        cnt = ((wid + 1) * n_chunks) // _NW - base

        pltpu.sync_copy(zero_hbm.at[pl.ds(s * band, band), :],
                        acc.at[pl.ds(s * band, band), :])
        plsc.subcore_barrier()

        def fire_idx(j):
            pltpu.async_copy(
                dst_hbm.at[pl.ds((base + j) * _CHUNK, _CHUNK)],
                idxb[j % _RING_S], semi[j % _RING_S])

        def wait_idx(j):
            pltpu.make_async_copy(
                dst_hbm.at[pl.ds((base + j) * _CHUNK, _CHUNK)],
                idxb[j % _RING_S], semi[j % _RING_S]).wait()

        def fire_msg(j):
            pltpu.async_copy(msg_hbm.at[pl.ds((base + j) * _CHUNK, _CHUNK), :],
                             bufs.at[j % _RING_S], sl[j % _RING_S])

        def wait_msg(j):
            pltpu.make_async_copy(
                msg_hbm.at[pl.ds((base + j) * _CHUNK, _CHUNK), :],
                bufs.at[j % _RING_S], sl[j % _RING_S]).wait()

        def wait_s(j):
            pltpu.make_async_copy(bufs.at[j % _RING_S], acc.at[idxb[j % _RING_S]],
                                  ss[j % _RING_S]).wait()

        for j in range(_RING_S):
            @pl.when(j < cnt)
            def _(j=j):
                fire_idx(j)
                fire_msg(j)
        for j in range(maxc):
            @pl.when(j < cnt)
            def _(j=j):
                wait_idx(j)
                wait_msg(j)
                pltpu.async_copy(bufs.at[j % _RING_S], acc.at[idxb[j % _RING_S]],
                                 ss[j % _RING_S], add=True)

                @pl.when(j + _RING_S < cnt)
                def _():
                    wait_s(j)
                    fire_idx(j + _RING_S)
                    fire_msg(j + _RING_S)

        for r in range(_RING_S):
            @pl.when(r < cnt)
            def _(r=r):
                wait_s(r)

        plsc.subcore_barrier()
        pltpu.sync_copy(acc.at[pl.ds(s * band, band), :],
                        out_hbm.at[pl.ds(c * n_acc + s * band, band), :])

    return k(msg, dst, zeros_rows)


def _edge_msg(hs, tT, W2T, B2T, tile):
    E = hs.shape[0]
    Hd = tT.shape[0]

    def body(hs_ref, t_ref, w2_ref, b2_ref, o_ref):
        hsT = hs_ref[...].T[:Hd, :]
        hsTb = hsT.astype(jnp.bfloat16)
        tTb = t_ref[...].astype(jnp.bfloat16)
        parts = [tTb[k:k + 1, :] * hsTb for k in range(Hd)]
        PT = jnp.concatenate(parts, axis=0)
        acc = jnp.dot(w2_ref[...], PT, preferred_element_type=jnp.float32)
        acc = acc + jnp.dot(b2_ref[...], hsT, preferred_element_type=jnp.float32)
        o_ref[...] = acc.T

    return pl.pallas_call(
        body,
        grid=(E // tile,),
        in_specs=[pl.BlockSpec((tile, 2 * Hd), lambda i: (i, 0)),
                  pl.BlockSpec((Hd, tile), lambda i: (0, i)),
                  pl.BlockSpec((Hd, Hd * Hd), lambda i: (0, 0)),
                  pl.BlockSpec((Hd, Hd), lambda i: (0, 0))],
        out_specs=pl.BlockSpec((tile, Hd), lambda i: (i, 0)),
        out_shape=jax.ShapeDtypeStruct((E, Hd), jnp.float32),
    )(hs, tT, W2T.astype(jnp.bfloat16), B2T)


def _gru(agg2, hid64, Wih, Whh, bih, bhh, convb, tile, n_pad):
    n_rows = agg2.shape[0] // 2
    Hd = hid64.shape[1]
    nb = n_rows // tile

    def body(a0_ref, a1_ref, h_ref, wih_ref, whh_ref, bih_ref, bhh_ref,
             cb_ref, o64_ref, o128_ref):
        m = jnp.maximum(a0_ref[...] + a1_ref[...] + cb_ref[...], 0.0)
        hv = h_ref[...]
        gi = jnp.dot(m, wih_ref[...], preferred_element_type=jnp.float32) + bih_ref[...]
        gh = jnp.dot(hv, whh_ref[...], preferred_element_type=jnp.float32) + bhh_ref[...]
        r = jax.nn.sigmoid(gi[:, :Hd] + gh[:, :Hd])
        z = jax.nn.sigmoid(gi[:, Hd:2 * Hd] + gh[:, Hd:2 * Hd])
        ng = jnp.tanh(gi[:, 2 * Hd:] + r * gh[:, 2 * Hd:])
        nh = (1.0 - z) * ng + z * hv
        o64_ref[...] = nh
        o128_ref[...] = jnp.concatenate(
            [nh, jnp.zeros((tile, Hd), jnp.float32)], axis=1)

    return pl.pallas_call(
        body,
        grid=(nb,),
        in_specs=[pl.BlockSpec((tile, Hd), lambda i: (i, 0)),
                  pl.BlockSpec((tile, Hd), lambda i: (i + nb, 0)),
                  pl.BlockSpec((tile, Hd), lambda i: (i, 0)),
                  pl.BlockSpec((Hd, 3 * Hd), lambda i: (0, 0)),
                  pl.BlockSpec((Hd, 3 * Hd), lambda i: (0, 0)),
                  pl.BlockSpec((1, 3 * Hd), lambda i: (0, 0)),
                  pl.BlockSpec((1, 3 * Hd), lambda i: (0, 0)),
                  pl.BlockSpec((1, Hd), lambda i: (0, 0))],
        out_specs=[pl.BlockSpec((tile, Hd), lambda i: (i, 0)),
                   pl.BlockSpec((tile, 2 * Hd), lambda i: (i, 0))],
        out_shape=[jax.ShapeDtypeStruct((n_pad, Hd), jnp.float32),
                   jax.ShapeDtypeStruct((n_pad, 2 * Hd), jnp.float32)],
    )(agg2, agg2, hid64, Wih, Whh, bih.reshape(1, -1), bhh.reshape(1, -1),
      convb.reshape(1, -1))


def _set2set(h, n2g_tiles, lstm, ro_W, ro_b, fc_W, fc_b, n_graphs):
    Hd = h.shape[1]
    n_tiles, _, tile = n2g_tiles.shape

    def body(h_ref, g_ref, wi0, wh0, bi0, bh0, wi1, wh1, bi1, bh1,
             wi2, wh2, bi2, bh2, row, rob, fcw, fcb, o_ref):
        iota_b = lax.broadcasted_iota(jnp.int32, (n_graphs, tile), 0)
        Wi = [wi0, wi1, wi2]
        Wh = [wh0, wh1, wh2]
        Bi = [bi0, bi1, bi2]
        Bh = [bh0, bh1, bh2]
        hs = [jnp.zeros((n_graphs, Hd), jnp.float32) for _ in range(3)]
        cs = [jnp.zeros((n_graphs, Hd), jnp.float32) for _ in range(3)]
        q_star = jnp.zeros((n_graphs, 2 * Hd), jnp.float32)
        dn0 = (((0,), (0,)), ((), ()))

        for _ in range(3):
            x = q_star
            for l in range(3):
                gates = (jnp.dot(x, Wi[l][...], preferred_element_type=jnp.float32)
                         + Bi[l][...]
                         + jnp.dot(hs[l], Wh[l][...], preferred_element_type=jnp.float32)
                         + Bh[l][...])
                ii = jax.nn.sigmoid(gates[:, :Hd])
                ff = jax.nn.sigmoid(gates[:, Hd:2 * Hd])
                gg = jnp.tanh(gates[:, 2 * Hd:3 * Hd])
                oo = jax.nn.sigmoid(gates[:, 3 * Hd:])
                cs[l] = ff * cs[l] + ii * gg
                hs[l] = oo * jnp.tanh(cs[l])
                x = hs[l]
            q = x

            def tile_data(j):
                ht = h_ref[pl.ds(j * tile, tile), :]
                gt = g_ref[j]
                MT = iota_b == gt
                MTf = MT.astype(jnp.float32)
                qnT = lax.dot_general(q, MTf, dn0,
                                      preferred_element_type=jnp.float32)
                eT = jnp.sum(ht.T * qnT, axis=0, keepdims=True)
                return ht, MT, MTf, eT

            def p1(j, m_col):
                _, MT, _, eT = tile_data(j)
                me = jnp.where(MT, eT, _NEG)
                return jnp.maximum(m_col, jnp.max(me, axis=1, keepdims=True))

            m_col = lax.fori_loop(0, n_tiles, p1,
                                  jnp.full((n_graphs, 1), _NEG))
            m_row = m_col.T

            def p2(j, carry):
                s_col, r_b = carry
                ht, _, MTf, eT = tile_data(j)
                mrow = jnp.dot(m_row, MTf, preferred_element_type=jnp.float32)
                exT = jnp.exp(eT - mrow)
                Mx = MTf * exT
                s_col = s_col + jnp.sum(Mx, axis=1, keepdims=True)
                r_b = r_b + jnp.dot(Mx, ht, preferred_element_type=jnp.float32)
                return s_col, r_b

            s_col, r_b = lax.fori_loop(
                0, n_tiles, p2,
                (jnp.zeros((n_graphs, 1), jnp.float32),
                 jnp.zeros((n_graphs, Hd), jnp.float32)))
            r = r_b / (s_col + 1e-12)
            q_star = jnp.concatenate([q, r], axis=1)

        gf = jnp.dot(q_star, row[...], preferred_element_type=jnp.float32) + rob[...]
        val = jnp.sum(gf * fcw[...], axis=1, keepdims=True) + fcb[...]
        o_ref[...] = jax.nn.sigmoid(val)

    args = [h, n2g_tiles]
    in_specs = [pl.BlockSpec(h.shape, lambda: (0, 0)),
                pl.BlockSpec(n2g_tiles.shape, lambda: (0, 0, 0))]
    for (Wi, Wh, bi, bh) in lstm:
        args += [Wi, Wh, bi.reshape(1, -1), bh.reshape(1, -1)]
        in_specs += [pl.BlockSpec(Wi.shape, lambda: (0, 0)),
                     pl.BlockSpec(Wh.shape, lambda: (0, 0)),
                     pl.BlockSpec((1, bi.shape[0]), lambda: (0, 0)),
                     pl.BlockSpec((1, bh.shape[0]), lambda: (0, 0))]
    args += [ro_W, ro_b.reshape(1, -1), fc_W.reshape(1, -1), fc_b.reshape(1, 1)]
    in_specs += [pl.BlockSpec(ro_W.shape, lambda: (0, 0)),
                 pl.BlockSpec((1, ro_b.shape[0]), lambda: (0, 0)),
                 pl.BlockSpec((1, fc_W.shape[0]), lambda: (0, 0)),
                 pl.BlockSpec((1, 1), lambda: (0, 0))]

    return pl.pallas_call(
        body,
        in_specs=in_specs,
        out_specs=pl.BlockSpec((n_graphs, 1), lambda: (0, 0)),
        out_shape=jax.ShapeDtypeStruct((n_graphs, 1), jnp.float32),
    )(*args)


def kernel(n_feats, e_feats, edge_index, node2graph, proj_W, proj_b, eW1, eb1,
           eW2, eb2, conv_b, gru_Wih, gru_Whh, gru_bih, gru_bhh,
           lstm_Wi0, lstm_Wh0, lstm_bi0, lstm_bh0,
           lstm_Wi1, lstm_Wh1, lstm_bi1, lstm_bh1,
           lstm_Wi2, lstm_Wh2, lstm_bi2, lstm_bh2,
           ro_W, ro_b, fc_W, fc_b):
    n_nodes, _ = n_feats.shape
    n_edges = e_feats.shape[0]
    Hd = proj_W.shape[1]
    n_graphs = 512
    n_pad = 10240

    src1 = edge_index[0]
    dst1 = edge_index[1]

    hidden64, hidden128 = _relu_mm2(n_feats, proj_W, proj_b, 1000, n_pad)
    tT = _relu_mm_t(e_feats, eW1, eb1, tile=1280)

    W2T = eW2.reshape(Hd * Hd, Hd).T
    B2T = eb2.reshape(Hd, Hd).T
    zeros_acc = jnp.zeros((n_nodes, Hd), jnp.float32)

    for _ in range(3):
        hs = _sc_gather(hidden128, src1)
        msg = _edge_msg(hs, tT, W2T, B2T, tile=1280)
        agg2 = _sc_scatter(msg, dst1, zeros_acc)
        hidden64, hidden128 = _gru(agg2, hidden64, gru_Wih, gru_Whh,
                                   gru_bih, gru_bhh, conv_b,
                                   tile=1000, n_pad=n_pad)

    lstm = [(lstm_Wi0, lstm_Wh0, lstm_bi0, lstm_bh0),
            (lstm_Wi1, lstm_Wh1, lstm_bi1, lstm_bh1),
            (lstm_Wi2, lstm_Wh2, lstm_bi2, lstm_bh2)]
    out = _set2set(hidden64, node2graph.reshape(10, 1, 1000), lstm,
                   ro_W, ro_b, fc_W, fc_b, n_graphs)
    return out[:, 0]


# speedup vs baseline: 1.0030x; 1.0030x over previous
import functools

import jax
import jax.numpy as jnp
from jax import lax
from jax.experimental import pallas as pl
from jax.experimental.pallas import tpu as pltpu
from jax.experimental.pallas import tpu_sc as plsc

_NC, _NS = 2, 16
_NW = _NC * _NS
_CHUNK = 128
_RING = 6
_RING_S = 8
_NEG = -1e30


def _sc_mesh():
    return plsc.VectorSubcoreMesh(
        core_axis_name="c", subcore_axis_name="s",
        num_cores=_NC, num_subcores=_NS)


def _relu_mm2(x, W, b, tile, n_pad):
    M, K = x.shape
    O = W.shape[1]

    def body(x_ref, w_ref, b_ref, o64_ref, o128_ref):
        acc = jnp.dot(x_ref[...], w_ref[...], preferred_element_type=jnp.float32)
        v = jnp.maximum(acc + b_ref[...], 0.0)
        o64_ref[...] = v
        o128_ref[...] = jnp.concatenate(
            [v, jnp.zeros((tile, O), jnp.float32)], axis=1)

    return pl.pallas_call(
        body,
        grid=(M // tile,),
        in_specs=[pl.BlockSpec((tile, K), lambda i: (i, 0)),
                  pl.BlockSpec((K, O), lambda i: (0, 0)),
                  pl.BlockSpec((1, O), lambda i: (0, 0))],
        out_specs=[pl.BlockSpec((tile, O), lambda i: (i, 0)),
                   pl.BlockSpec((tile, 2 * O), lambda i: (i, 0))],
        out_shape=[jax.ShapeDtypeStruct((n_pad, O), jnp.float32),
                   jax.ShapeDtypeStruct((n_pad, 2 * O), jnp.float32)],
    )(x, W, b.reshape(1, O))


def _relu_mm_t(x, W, b, tile):
    M, K = x.shape
    O = W.shape[1]

    def body(x_ref, w_ref, b_ref, o_ref):
        acc = jnp.dot(x_ref[...], w_ref[...], preferred_element_type=jnp.float32)
        o_ref[...] = jnp.maximum(acc + b_ref[...], 0.0).T

    return pl.pallas_call(
        body,
        grid=(M // tile,),
        in_specs=[pl.BlockSpec((tile, K), lambda i: (i, 0)),
                  pl.BlockSpec((K, O), lambda i: (0, 0)),
                  pl.BlockSpec((1, O), lambda i: (0, 0))],
        out_specs=pl.BlockSpec((O, tile), lambda i: (0, i)),
        out_shape=jax.ShapeDtypeStruct((O, M), jnp.float32),
    )(x, W, b.reshape(1, O))


def _sc_gather(table, idx):
    n_rows, Hd = table.shape
    (E,) = idx.shape
    n_chunks = E // _CHUNK
    maxc = -(-n_chunks // _NW)

    @functools.partial(
        pl.kernel,
        out_type=jax.ShapeDtypeStruct((E, Hd), jnp.float32),
        mesh=_sc_mesh(),
        scratch_types=[
            [pltpu.VMEM((_CHUNK,), jnp.int32)] * _RING,
            pltpu.VMEM((_RING, _CHUNK, Hd), jnp.float32),
            [pltpu.SemaphoreType.DMA] * _RING,
        ] + [pltpu.SemaphoreType.DMA] * _RING,
    )
    def k(table_hbm, idx_hbm, out_hbm, idxb, rows, semi, *semg):
        wid = lax.axis_index("s") * _NC + lax.axis_index("c")
        base = (wid * n_chunks) // _NW
        cnt = ((wid + 1) * n_chunks) // _NW - base

        def fire_idx(j):
            pltpu.async_copy(
                idx_hbm.at[pl.ds((base + j) * _CHUNK, _CHUNK)],
                idxb[j % _RING], semi[j % _RING])

        def wait_idx(j):
            pltpu.make_async_copy(
                idx_hbm.at[pl.ds((base + j) * _CHUNK, _CHUNK)],
                idxb[j % _RING], semi[j % _RING]).wait()

        def fire_g(j):
            pltpu.async_copy(table_hbm.at[idxb[j % _RING]],
                             rows.at[j % _RING], semg[j % _RING])

        def wait_g(j):
            pltpu.make_async_copy(table_hbm.at[idxb[j % _RING]],
                                  rows.at[j % _RING], semg[j % _RING]).wait()

        for j in range(_RING):
            pl.when(j < cnt)(lambda j=j: fire_idx(j))
        for j in range(_RING - 1):
            @pl.when(j < cnt)
            def _(j=j):
                wait_idx(j)
                fire_g(j)
        for j in range(maxc):
            @pl.when(j < cnt)
            def _(j=j):
                wait_g(j)
                pltpu.sync_copy(rows.at[j % _RING],
                                out_hbm.at[pl.ds((base + j) * _CHUNK, _CHUNK), :])
                pl.when(j + _RING < cnt)(lambda j=j: fire_idx(j + _RING))

                @pl.when(j + _RING - 1 < cnt)
                def _():
                    wait_idx(j + _RING - 1)
                    fire_g(j + _RING - 1)

    return k(table, idx)


def _sc_scatter(msg, dst2, zeros_rows):
    E, Hd = msg.shape
    n_acc = zeros_rows.shape[0]
    n_chunks = dst2.shape[0]
    maxc = -(-n_chunks // _NW)
    band = n_acc // _NS

    @functools.partial(
        pl.kernel,
        out_type=jax.ShapeDtypeStruct((2 * n_acc, Hd), jnp.float32),
        mesh=_sc_mesh(),
        compiler_params=pltpu.CompilerParams(use_tc_tiling_on_sc=False),
        scratch_types=[
            pltpu.VMEM((maxc, _CHUNK), jnp.int32),
            pltpu.VMEM((_RING_S, _CHUNK, Hd), jnp.float32),
            pltpu.VMEM_SHARED((n_acc, Hd), jnp.float32),
        ] + [pltpu.SemaphoreType.DMA] * (2 * _RING_S),
    )
    def k(msg_hbm, dst_hbm, zero_hbm, out_hbm, idx2d, bufs, acc, *sems):
        sl = sems[:_RING_S]
        ss = sems[_RING_S:]
        c = lax.axis_index("c")
        s = lax.axis_index("s")
        wid = s * _NC + c
        base = (wid * n_chunks) // _NW
        cnt = ((wid + 1) * n_chunks) // _NW - base

        pltpu.sync_copy(zero_hbm.at[pl.ds(s * band, band), :],
                        acc.at[pl.ds(s * band, band), :])
        pltpu.sync_copy(dst_hbm.at[pl.ds(base, maxc)], idx2d)
        plsc.subcore_barrier()

        def load(j):
            pltpu.async_copy(msg_hbm.at[pl.ds((base + j) * _CHUNK, _CHUNK), :],
                             bufs.at[j % _RING_S], sl[j % _RING_S])

        for j in range(_RING_S):
            pl.when(j < cnt)(lambda j=j: load(j))
        for j in range(maxc):
            @pl.when(j < cnt)
            def _(j=j):
                pltpu.make_async_copy(
                    msg_hbm.at[pl.ds((base + j) * _CHUNK, _CHUNK), :],
                    bufs.at[j % _RING_S], sl[j % _RING_S]).wait()
                pltpu.async_copy(bufs.at[j % _RING_S], acc.at[idx2d.at[j]],
                                 ss[j % _RING_S], add=True)

                @pl.when(j + _RING_S < cnt)
                def _():
                    pltpu.make_async_copy(bufs.at[j % _RING_S],
                                          acc.at[idx2d.at[j]],
                                          ss[j % _RING_S]).wait()
                    load(j + _RING_S)

        for r in range(_RING_S):
            @pl.when(r < cnt)
            def _(r=r):
                pltpu.make_async_copy(bufs.at[r], acc.at[idx2d.at[r]],
                                      ss[r]).wait()

        plsc.subcore_barrier()
        pltpu.sync_copy(acc.at[pl.ds(s * band, band), :],
                        out_hbm.at[pl.ds(c * n_acc + s * band, band), :])

    return k(msg, dst2, zeros_rows)


def _edge_msg(hs, tT, W2T, B2T, tile):
    E = hs.shape[0]
    Hd = tT.shape[0]

    def body(hs_ref, t_ref, w2_ref, b2_ref, o_ref):
        hsT = hs_ref[...].T[:Hd, :]
        hsTb = hsT.astype(jnp.bfloat16)
        tTb = t_ref[...].astype(jnp.bfloat16)
        parts = [tTb[k:k + 1, :] * hsTb for k in range(Hd)]
        PT = jnp.concatenate(parts, axis=0)
        acc = jnp.dot(w2_ref[...], PT, preferred_element_type=jnp.float32)
        acc = acc + jnp.dot(b2_ref[...], hsT, preferred_element_type=jnp.float32)
        o_ref[...] = acc.T

    return pl.pallas_call(
        body,
        grid=(E // tile,),
        in_specs=[pl.BlockSpec((tile, 2 * Hd), lambda i: (i, 0)),
                  pl.BlockSpec((Hd, tile), lambda i: (0, i)),
                  pl.BlockSpec((Hd, Hd * Hd), lambda i: (0, 0)),
                  pl.BlockSpec((Hd, Hd), lambda i: (0, 0))],
        out_specs=pl.BlockSpec((tile, Hd), lambda i: (i, 0)),
        out_shape=jax.ShapeDtypeStruct((E, Hd), jnp.float32),
    )(hs, tT, W2T.astype(jnp.bfloat16), B2T)


def _gru(agg2, hid64, Wih, Whh, bih, bhh, convb, tile, n_pad):
    n_rows = agg2.shape[0] // 2
    Hd = hid64.shape[1]
    nb = n_rows // tile

    def body(a0_ref, a1_ref, h_ref, wih_ref, whh_ref, bih_ref, bhh_ref,
             cb_ref, o64_ref, o128_ref):
        m = jnp.maximum(a0_ref[...] + a1_ref[...] + cb_ref[...], 0.0)
        hv = h_ref[...]
        gi = jnp.dot(m, wih_ref[...], preferred_element_type=jnp.float32) + bih_ref[...]
        gh = jnp.dot(hv, whh_ref[...], preferred_element_type=jnp.float32) + bhh_ref[...]
        r = jax.nn.sigmoid(gi[:, :Hd] + gh[:, :Hd])
        z = jax.nn.sigmoid(gi[:, Hd:2 * Hd] + gh[:, Hd:2 * Hd])
        ng = jnp.tanh(gi[:, 2 * Hd:] + r * gh[:, 2 * Hd:])
        nh = (1.0 - z) * ng + z * hv
        o64_ref[...] = nh
        o128_ref[...] = jnp.concatenate(
            [nh, jnp.zeros((tile, Hd), jnp.float32)], axis=1)

    return pl.pallas_call(
        body,
        grid=(nb,),
        in_specs=[pl.BlockSpec((tile, Hd), lambda i: (i, 0)),
                  pl.BlockSpec((tile, Hd), lambda i: (i + nb, 0)),
                  pl.BlockSpec((tile, Hd), lambda i: (i, 0)),
                  pl.BlockSpec((Hd, 3 * Hd), lambda i: (0, 0)),
                  pl.BlockSpec((Hd, 3 * Hd), lambda i: (0, 0)),
                  pl.BlockSpec((1, 3 * Hd), lambda i: (0, 0)),
                  pl.BlockSpec((1, 3 * Hd), lambda i: (0, 0)),
                  pl.BlockSpec((1, Hd), lambda i: (0, 0))],
        out_specs=[pl.BlockSpec((tile, Hd), lambda i: (i, 0)),
                   pl.BlockSpec((tile, 2 * Hd), lambda i: (i, 0))],
        out_shape=[jax.ShapeDtypeStruct((n_pad, Hd), jnp.float32),
                   jax.ShapeDtypeStruct((n_pad, 2 * Hd), jnp.float32)],
    )(agg2, agg2, hid64, Wih, Whh, bih.reshape(1, -1), bhh.reshape(1, -1),
      convb.reshape(1, -1))


def _set2set(h, n2g_tiles, lstm, ro_W, ro_b, fc_W, fc_b, n_graphs):
    Hd = h.shape[1]
    n_tiles, _, tile = n2g_tiles.shape

    def body(h_ref, g_ref, wi0, wh0, bi0, bh0, wi1, wh1, bi1, bh1,
             wi2, wh2, bi2, bh2, row, rob, fcw, fcb, o_ref):
        iota_b = lax.broadcasted_iota(jnp.int32, (n_graphs, tile), 0)
        Wi = [wi0, wi1, wi2]
        Wh = [wh0, wh1, wh2]
        Bi = [bi0, bi1, bi2]
        Bh = [bh0, bh1, bh2]
        hs = [jnp.zeros((n_graphs, Hd), jnp.float32) for _ in range(3)]
        cs = [jnp.zeros((n_graphs, Hd), jnp.float32) for _ in range(3)]
        q_star = jnp.zeros((n_graphs, 2 * Hd), jnp.float32)
        dn0 = (((0,), (0,)), ((), ()))

        for _ in range(3):
            x = q_star
            for l in range(3):
                gates = (jnp.dot(x, Wi[l][...], preferred_element_type=jnp.float32)
                         + Bi[l][...]
                         + jnp.dot(hs[l], Wh[l][...], preferred_element_type=jnp.float32)
                         + Bh[l][...])
                ii = jax.nn.sigmoid(gates[:, :Hd])
                ff = jax.nn.sigmoid(gates[:, Hd:2 * Hd])
                gg = jnp.tanh(gates[:, 2 * Hd:3 * Hd])
                oo = jax.nn.sigmoid(gates[:, 3 * Hd:])
                cs[l] = ff * cs[l] + ii * gg
                hs[l] = oo * jnp.tanh(cs[l])
                x = hs[l]
            q = x

            def tile_data(j):
                ht = h_ref[pl.ds(j * tile, tile), :]
                gt = g_ref[j]
                MT = iota_b == gt
                MTf = MT.astype(jnp.float32)
                qnT = lax.dot_general(q, MTf, dn0,
                                      preferred_element_type=jnp.float32)
                eT = jnp.sum(ht.T * qnT, axis=0, keepdims=True)
                return ht, MT, MTf, eT

            def p1(j, m_col):
                _, MT, _, eT = tile_data(j)
                me = jnp.where(MT, eT, _NEG)
                return jnp.maximum(m_col, jnp.max(me, axis=1, keepdims=True))

            m_col = lax.fori_loop(0, n_tiles, p1,
                                  jnp.full((n_graphs, 1), _NEG))
            m_row = m_col.T

            def p2(j, carry):
                s_col, r_b = carry
                ht, _, MTf, eT = tile_data(j)
                mrow = jnp.dot(m_row, MTf, preferred_element_type=jnp.float32)
                exT = jnp.exp(eT - mrow)
                Mx = MTf * exT
                s_col = s_col + jnp.sum(Mx, axis=1, keepdims=True)
                r_b = r_b + jnp.dot(Mx, ht, preferred_element_type=jnp.float32)
                return s_col, r_b

            s_col, r_b = lax.fori_loop(
                0, n_tiles, p2,
                (jnp.zeros((n_graphs, 1), jnp.float32),
                 jnp.zeros((n_graphs, Hd), jnp.float32)))
            r = r_b / (s_col + 1e-12)
            q_star = jnp.concatenate([q, r], axis=1)

        gf = jnp.dot(q_star, row[...], preferred_element_type=jnp.float32) + rob[...]
        val = jnp.sum(gf * fcw[...], axis=1, keepdims=True) + fcb[...]
        o_ref[...] = jax.nn.sigmoid(val)

    args = [h, n2g_tiles]
    in_specs = [pl.BlockSpec(h.shape, lambda: (0, 0)),
                pl.BlockSpec(n2g_tiles.shape, lambda: (0, 0, 0))]
    for (Wi, Wh, bi, bh) in lstm:
        args += [Wi, Wh, bi.reshape(1, -1), bh.reshape(1, -1)]
        in_specs += [pl.BlockSpec(Wi.shape, lambda: (0, 0)),
                     pl.BlockSpec(Wh.shape, lambda: (0, 0)),
                     pl.BlockSpec((1, bi.shape[0]), lambda: (0, 0)),
                     pl.BlockSpec((1, bh.shape[0]), lambda: (0, 0))]
    args += [ro_W, ro_b.reshape(1, -1), fc_W.reshape(1, -1), fc_b.reshape(1, 1)]
    in_specs += [pl.BlockSpec(ro_W.shape, lambda: (0, 0)),
                 pl.BlockSpec((1, ro_b.shape[0]), lambda: (0, 0)),
                 pl.BlockSpec((1, fc_W.shape[0]), lambda: (0, 0)),
                 pl.BlockSpec((1, 1), lambda: (0, 0))]

    return pl.pallas_call(
        body,
        in_specs=in_specs,
        out_specs=pl.BlockSpec((n_graphs, 1), lambda: (0, 0)),
        out_shape=jax.ShapeDtypeStruct((n_graphs, 1), jnp.float32),
    )(*args)


def kernel(n_feats, e_feats, edge_index, node2graph, proj_W, proj_b, eW1, eb1,
           eW2, eb2, conv_b, gru_Wih, gru_Whh, gru_bih, gru_bhh,
           lstm_Wi0, lstm_Wh0, lstm_bi0, lstm_bh0,
           lstm_Wi1, lstm_Wh1, lstm_bi1, lstm_bh1,
           lstm_Wi2, lstm_Wh2, lstm_bi2, lstm_bh2,
           ro_W, ro_b, fc_W, fc_b):
    n_nodes, _ = n_feats.shape
    n_edges = e_feats.shape[0]
    Hd = proj_W.shape[1]
    n_graphs = 512
    n_pad = 10240

    src1 = edge_index[0]
    dst2 = edge_index[1].reshape(n_edges // _CHUNK, _CHUNK)

    hidden64, hidden128 = _relu_mm2(n_feats, proj_W, proj_b, 1000, n_pad)
    tT = _relu_mm_t(e_feats, eW1, eb1, tile=1280)

    W2T = eW2.reshape(Hd * Hd, Hd).T
    B2T = eb2.reshape(Hd, Hd).T
    zeros_acc = jnp.zeros((n_nodes, Hd), jnp.float32)

    for _ in range(3):
        hs = _sc_gather(hidden128, src1)
        msg = _edge_msg(hs, tT, W2T, B2T, tile=1280)
        agg2 = _sc_scatter(msg, dst2, zeros_acc)
        hidden64, hidden128 = _gru(agg2, hidden64, gru_Wih, gru_Whh,
                                   gru_bih, gru_bhh, conv_b,
                                   tile=1000, n_pad=n_pad)

    lstm = [(lstm_Wi0, lstm_Wh0, lstm_bi0, lstm_bh0),
            (lstm_Wi1, lstm_Wh1, lstm_bi1, lstm_bh1),
            (lstm_Wi2, lstm_Wh2, lstm_bi2, lstm_bh2)]
    out = _set2set(hidden64, node2graph.reshape(10, 1, 1000), lstm,
                   ro_W, ro_b, fc_W, fc_b, n_graphs)
    return out[:, 0]
